# baseline (speedup 1.0000x reference)
import functools

import jax
import jax.numpy as jnp
from jax import lax
from jax.experimental import pallas as pl
from jax.experimental.pallas import tpu as pltpu
from jax.experimental.pallas import tpu_sc as plsc

NUM_EMB = 1000000
B = 16384
SEQ = 20
D = 64
LANES = 16
NC, NS = 2, 16
NW = NC * NS
BPW = B // NW
G = 16
NCHUNK = BPW // G
IDX_PER_CHUNK = G * SEQ
GW = 64
NGATHER = IDX_PER_CHUNK // GW


def _tc_widen(tableT):
    BLK = 8192

    def body(t_ref, eye_ref, o_ref):
        tt = lax.dot_general(
            t_ref[...], eye_ref[...], (((0,), (0,)), ((), ())),
            preferred_element_type=jnp.float32,
        )
        o_ref[...] = jnp.concatenate([tt, tt], axis=1)

    return pl.pallas_call(
        body,
        grid=(pl.cdiv(NUM_EMB, BLK),),
        in_specs=[pl.BlockSpec((D, BLK), lambda i: (0, i)),
                  pl.BlockSpec((D, D), lambda i: (0, 0))],
        out_specs=pl.BlockSpec((BLK, 2 * D), lambda i: (i, 0)),
        out_shape=jax.ShapeDtypeStruct((NUM_EMB, 2 * D), jnp.float32),
        compiler_params=pltpu.CompilerParams(
            dimension_semantics=("parallel",)),
    )(tableT, jnp.eye(D, dtype=jnp.float32))


def _sc_pool(idx_flat, table_wide):
    mesh = plsc.VectorSubcoreMesh(core_axis_name="c", subcore_axis_name="s")

    @functools.partial(
        pl.kernel,
        mesh=mesh,
        out_type=jax.ShapeDtypeStruct((B, D), jnp.float32),
        scratch_types=[
            pltpu.VMEM((BPW * SEQ,), jnp.int32),
            pltpu.VMEM((IDX_PER_CHUNK, 2 * D), jnp.float32),
            pltpu.VMEM((IDX_PER_CHUNK, 2 * D), jnp.float32),
            pltpu.VMEM((G, D), jnp.float32),
            pltpu.SemaphoreType.DMA,
            pltpu.SemaphoreType.DMA,
        ],
    )
    def sc_kernel(idx_hbm, table_hbm, out_hbm, idx_v, rows_a, rows_b,
                  pool_v, sem_a, sem_b):
        wid = lax.axis_index("s") * NC + lax.axis_index("c")
        ibase = wid * (BPW * SEQ)
        pltpu.sync_copy(idx_hbm.at[pl.ds(ibase, BPW * SEQ)], idx_v)

        def fire(ci, rows, sem):
            for j in range(NGATHER):
                pltpu.async_copy(
                    table_hbm.at[idx_v.at[pl.ds(ci * IDX_PER_CHUNK + j * GW, GW)]],
                    rows.at[pl.ds(j * GW, GW)],
                    sem,
                )

        def drain(rows, sem):
            pltpu.make_async_copy(
                table_hbm.at[pl.ds(0, IDX_PER_CHUNK)], rows, sem).wait()

        def compute(ci, rows):
            @pl.loop(0, G)
            def _group(g):
                base = g * SEQ
                for c in range(D // LANES):
                    sl = pl.ds(c * LANES, LANES)
                    acc = rows[base, sl]
                    for j in range(1, SEQ):
                        acc = acc + rows[base + j, sl]
                    pool_v[g, sl] = acc

            pltpu.sync_copy(
                pool_v, out_hbm.at[pl.ds(wid * BPW + ci * G, G)])

        fire(0, rows_a, sem_a)

        @pl.loop(0, NCHUNK // 2)
        def _pair(cp):
            ci0 = 2 * cp
            fire(ci0 + 1, rows_b, sem_b)
            drain(rows_a, sem_a)
            compute(ci0, rows_a)

            @pl.when(cp < NCHUNK // 2 - 1)
            def _():
                fire(ci0 + 2, rows_a, sem_a)

            drain(rows_b, sem_b)
            compute(ci0 + 1, rows_b)

    return sc_kernel(idx_flat, table_wide)


def _tc_head(pooled, W, b_scaled):
    BB = 2048

    def body(x_ref, w_ref, b_ref, o_ref):
        z = lax.dot_general(
            x_ref[...], w_ref[...], (((1,), (0,)), ((), ())),
            preferred_element_type=jnp.float32,
            precision=lax.Precision.HIGHEST,
        )
        z = z + b_ref[0:1, :]
        s = jnp.sum(z * z, axis=1, keepdims=True)
        o_ref[...] = z * lax.rsqrt(s)

    return pl.pallas_call(
        body,
        grid=(B // BB,),
        in_specs=[
            pl.BlockSpec((BB, D), lambda i: (i, 0)),
            pl.BlockSpec((D, D), lambda i: (0, 0)),
            pl.BlockSpec((8, D), lambda i: (0, 0)),
        ],
        out_specs=pl.BlockSpec((BB, D), lambda i: (i, 0)),
        out_shape=jax.ShapeDtypeStruct((B, D), jnp.float32),
        compiler_params=pltpu.CompilerParams(
            dimension_semantics=("parallel",)),
    )(pooled, W, b_scaled)


def kernel(inputs, table, W, b):
    idx_flat = inputs.reshape(-1).astype(jnp.int32)
    table_wide = _tc_widen(table.T)
    pooled = _sc_pool(idx_flat, table_wide)
    b_scaled = jnp.broadcast_to((SEQ * b).reshape(1, D), (8, D))
    return _tc_head(pooled, W, b_scaled)


# speedup vs baseline: 1.0048x; 1.0048x over previous
import functools

import jax
import jax.numpy as jnp
from jax import lax
from jax.experimental import pallas as pl
from jax.experimental.pallas import tpu as pltpu
from jax.experimental.pallas import tpu_sc as plsc

NUM_EMB = 1000000
B = 16384
SEQ = 20
D = 64
LANES = 16
NC, NS = 2, 16
NW = NC * NS
BPW = B // NW
G = 16
NCHUNK = BPW // G
IDX_PER_CHUNK = G * SEQ
GW = 64
NGATHER = IDX_PER_CHUNK // GW


def _tc_widen(tableT):
    BLK = 8192

    def body(t_ref, o_ref):
        tt = jnp.transpose(t_ref[...], (1, 0))
        o_ref[...] = jnp.concatenate([tt, tt], axis=1)

    return pl.pallas_call(
        body,
        grid=(pl.cdiv(NUM_EMB, BLK),),
        in_specs=[pl.BlockSpec((D, BLK), lambda i: (0, i))],
        out_specs=pl.BlockSpec((BLK, 2 * D), lambda i: (i, 0)),
        out_shape=jax.ShapeDtypeStruct((NUM_EMB, 2 * D), jnp.float32),
        compiler_params=pltpu.CompilerParams(
            dimension_semantics=("parallel",)),
    )(tableT)


def _sc_pool(idx_flat, table_wide):
    mesh = plsc.VectorSubcoreMesh(core_axis_name="c", subcore_axis_name="s")

    @functools.partial(
        pl.kernel,
        mesh=mesh,
        out_type=jax.ShapeDtypeStruct((B, D), jnp.float32),
        scratch_types=[
            pltpu.VMEM((BPW * SEQ,), jnp.int32),
            pltpu.VMEM((IDX_PER_CHUNK, 2 * D), jnp.float32),
            pltpu.VMEM((IDX_PER_CHUNK, 2 * D), jnp.float32),
            pltpu.VMEM((G, D), jnp.float32),
            pltpu.SemaphoreType.DMA,
            pltpu.SemaphoreType.DMA,
        ],
    )
    def sc_kernel(idx_hbm, table_hbm, out_hbm, idx_v, rows_a, rows_b,
                  pool_v, sem_a, sem_b):
        wid = lax.axis_index("s") * NC + lax.axis_index("c")
        ibase = wid * (BPW * SEQ)
        pltpu.sync_copy(idx_hbm.at[pl.ds(ibase, BPW * SEQ)], idx_v)

        def fire(ci, rows, sem):
            for j in range(NGATHER):
                pltpu.async_copy(
                    table_hbm.at[idx_v.at[pl.ds(ci * IDX_PER_CHUNK + j * GW, GW)]],
                    rows.at[pl.ds(j * GW, GW)],
                    sem,
                )

        def drain(rows, sem):
            pltpu.make_async_copy(
                table_hbm.at[pl.ds(0, IDX_PER_CHUNK)], rows, sem).wait()

        def compute(ci, rows):
            @pl.loop(0, G)
            def _group(g):
                base = g * SEQ
                for c in range(D // LANES):
                    sl = pl.ds(c * LANES, LANES)
                    acc = rows[base, sl]
                    for j in range(1, SEQ):
                        acc = acc + rows[base + j, sl]
                    pool_v[g, sl] = acc

            pltpu.sync_copy(
                pool_v, out_hbm.at[pl.ds(wid * BPW + ci * G, G)])

        fire(0, rows_a, sem_a)

        @pl.loop(0, NCHUNK // 2)
        def _pair(cp):
            ci0 = 2 * cp
            fire(ci0 + 1, rows_b, sem_b)
            drain(rows_a, sem_a)
            compute(ci0, rows_a)

            @pl.when(cp < NCHUNK // 2 - 1)
            def _():
                fire(ci0 + 2, rows_a, sem_a)

            drain(rows_b, sem_b)
            compute(ci0 + 1, rows_b)

    return sc_kernel(idx_flat, table_wide)


def _tc_head(pooled, W, b_scaled):
    BB = 2048

    def body(x_ref, w_ref, b_ref, o_ref):
        z = lax.dot_general(
            x_ref[...], w_ref[...], (((1,), (0,)), ((), ())),
            preferred_element_type=jnp.float32,
            precision=lax.Precision.HIGHEST,
        )
        z = z + b_ref[0:1, :]
        s = jnp.sum(z * z, axis=1, keepdims=True)
        o_ref[...] = z * lax.rsqrt(s)

    return pl.pallas_call(
        body,
        grid=(B // BB,),
        in_specs=[
            pl.BlockSpec((BB, D), lambda i: (i, 0)),
            pl.BlockSpec((D, D), lambda i: (0, 0)),
            pl.BlockSpec((8, D), lambda i: (0, 0)),
        ],
        out_specs=pl.BlockSpec((BB, D), lambda i: (i, 0)),
        out_shape=jax.ShapeDtypeStruct((B, D), jnp.float32),
        compiler_params=pltpu.CompilerParams(
            dimension_semantics=("parallel",)),
    )(pooled, W, b_scaled)


def kernel(inputs, table, W, b):
    idx_flat = inputs.reshape(-1).astype(jnp.int32)
    table_wide = _tc_widen(table.T)
    pooled = _sc_pool(idx_flat, table_wide)
    b_scaled = jnp.broadcast_to((SEQ * b).reshape(1, D), (8, D))
    return _tc_head(pooled, W, b_scaled)


# speedup vs baseline: 1.0920x; 1.0867x over previous
import functools

import jax
import jax.numpy as jnp
from jax import lax
from jax.experimental import pallas as pl
from jax.experimental.pallas import tpu as pltpu
from jax.experimental.pallas import tpu_sc as plsc

NUM_EMB = 1000000
B = 16384
SEQ = 20
D = 64
LANES = 16
NC, NS = 2, 16
NW = NC * NS
BPW = B // NW
G = 16
NCHUNK = BPW // G
IDX_PER_CHUNK = G * SEQ
GW = 64
NGATHER = IDX_PER_CHUNK // GW


def _tc_widen(tableT):
    BLK = 16384

    def body(t_ref, o_ref):
        tt = jnp.transpose(t_ref[...], (1, 0))
        o_ref[...] = jnp.concatenate([tt, tt], axis=1)

    return pl.pallas_call(
        body,
        grid=(pl.cdiv(NUM_EMB, BLK),),
        in_specs=[pl.BlockSpec((D, BLK), lambda i: (0, i))],
        out_specs=pl.BlockSpec((BLK, 2 * D), lambda i: (i, 0)),
        out_shape=jax.ShapeDtypeStruct((NUM_EMB, 2 * D), jnp.float32),
        compiler_params=pltpu.CompilerParams(
            dimension_semantics=("parallel",)),
    )(tableT)


def _sc_pool(idx_flat, table_wide):
    mesh = plsc.VectorSubcoreMesh(core_axis_name="c", subcore_axis_name="s")

    @functools.partial(
        pl.kernel,
        mesh=mesh,
        out_type=jax.ShapeDtypeStruct((B, D), jnp.float32),
        scratch_types=[
            pltpu.VMEM((BPW * SEQ,), jnp.int32),
            pltpu.VMEM((IDX_PER_CHUNK, 2 * D), jnp.float32),
            pltpu.VMEM((IDX_PER_CHUNK, 2 * D), jnp.float32),
            pltpu.VMEM((G, D), jnp.float32),
            pltpu.SemaphoreType.DMA,
            pltpu.SemaphoreType.DMA,
        ],
    )
    def sc_kernel(idx_hbm, table_hbm, out_hbm, idx_v, rows_a, rows_b,
                  pool_v, sem_a, sem_b):
        wid = lax.axis_index("s") * NC + lax.axis_index("c")
        ibase = wid * (BPW * SEQ)
        pltpu.sync_copy(idx_hbm.at[pl.ds(ibase, BPW * SEQ)], idx_v)

        def fire(ci, rows, sem):
            for j in range(NGATHER):
                pltpu.async_copy(
                    table_hbm.at[idx_v.at[pl.ds(ci * IDX_PER_CHUNK + j * GW, GW)]],
                    rows.at[pl.ds(j * GW, GW)],
                    sem,
                )

        def drain(rows, sem):
            pltpu.make_async_copy(
                table_hbm.at[pl.ds(0, IDX_PER_CHUNK)], rows, sem).wait()

        def compute(ci, rows):
            @pl.loop(0, G)
            def _group(g):
                base = g * SEQ
                for c in range(D // LANES):
                    sl = pl.ds(c * LANES, LANES)
                    acc = rows[base, sl]
                    for j in range(1, SEQ):
                        acc = acc + rows[base + j, sl]
                    pool_v[g, sl] = acc

            pltpu.sync_copy(
                pool_v, out_hbm.at[pl.ds(wid * BPW + ci * G, G)])

        fire(0, rows_a, sem_a)

        @pl.loop(0, NCHUNK // 2)
        def _pair(cp):
            ci0 = 2 * cp
            fire(ci0 + 1, rows_b, sem_b)
            drain(rows_a, sem_a)
            compute(ci0, rows_a)

            @pl.when(cp < NCHUNK // 2 - 1)
            def _():
                fire(ci0 + 2, rows_a, sem_a)

            drain(rows_b, sem_b)
            compute(ci0 + 1, rows_b)

    return sc_kernel(idx_flat, table_wide)


def _tc_head(pooled, W, b_scaled):
    BB = 2048

    def body(x_ref, w_ref, b_ref, o_ref):
        z = lax.dot_general(
            x_ref[...], w_ref[...], (((1,), (0,)), ((), ())),
            preferred_element_type=jnp.float32,
            precision=lax.Precision.HIGHEST,
        )
        z = z + b_ref[0:1, :]
        s = jnp.sum(z * z, axis=1, keepdims=True)
        o_ref[...] = z * lax.rsqrt(s)

    return pl.pallas_call(
        body,
        grid=(B // BB,),
        in_specs=[
            pl.BlockSpec((BB, D), lambda i: (i, 0)),
            pl.BlockSpec((D, D), lambda i: (0, 0)),
            pl.BlockSpec((8, D), lambda i: (0, 0)),
        ],
        out_specs=pl.BlockSpec((BB, D), lambda i: (i, 0)),
        out_shape=jax.ShapeDtypeStruct((B, D), jnp.float32),
        compiler_params=pltpu.CompilerParams(
            dimension_semantics=("parallel",)),
    )(pooled, W, b_scaled)


def kernel(inputs, table, W, b):
    idx_flat = inputs.reshape(-1).astype(jnp.int32)
    table_wide = _tc_widen(table.T)
    pooled = _sc_pool(idx_flat, table_wide)
    b_scaled = jnp.broadcast_to((SEQ * b).reshape(1, D), (8, D))
    return _tc_head(pooled, W, b_scaled)


# speedup vs baseline: 1.1191x; 1.0249x over previous
import functools

import jax
import jax.numpy as jnp
from jax import lax
from jax.experimental import pallas as pl
from jax.experimental.pallas import tpu as pltpu
from jax.experimental.pallas import tpu_sc as plsc

NUM_EMB = 1000000
B = 16384
SEQ = 20
D = 64
LANES = 16
NC, NS = 2, 16
NW = NC * NS
BPW = B // NW
G = 16
NCHUNK = BPW // G
IDX_PER_CHUNK = G * SEQ
GW = 64
NGATHER = IDX_PER_CHUNK // GW


def _tc_widen(tableT):
    BLK = 24576

    def body(t_ref, o_ref):
        tt = jnp.transpose(t_ref[...], (1, 0))
        o_ref[...] = jnp.concatenate([tt, tt], axis=1)

    return pl.pallas_call(
        body,
        grid=(pl.cdiv(NUM_EMB, BLK),),
        in_specs=[pl.BlockSpec((D, BLK), lambda i: (0, i))],
        out_specs=pl.BlockSpec((BLK, 2 * D), lambda i: (i, 0)),
        out_shape=jax.ShapeDtypeStruct((NUM_EMB, 2 * D), jnp.float32),
        compiler_params=pltpu.CompilerParams(
            dimension_semantics=("parallel",)),
    )(tableT)


def _sc_pool(idx_flat, table_wide):
    mesh = plsc.VectorSubcoreMesh(core_axis_name="c", subcore_axis_name="s")

    @functools.partial(
        pl.kernel,
        mesh=mesh,
        out_type=jax.ShapeDtypeStruct((B, D), jnp.float32),
        scratch_types=[
            pltpu.VMEM((BPW * SEQ,), jnp.int32),
            pltpu.VMEM((IDX_PER_CHUNK, 2 * D), jnp.float32),
            pltpu.VMEM((IDX_PER_CHUNK, 2 * D), jnp.float32),
            pltpu.VMEM((G, D), jnp.float32),
            pltpu.SemaphoreType.DMA,
            pltpu.SemaphoreType.DMA,
        ],
    )
    def sc_kernel(idx_hbm, table_hbm, out_hbm, idx_v, rows_a, rows_b,
                  pool_v, sem_a, sem_b):
        wid = lax.axis_index("s") * NC + lax.axis_index("c")
        ibase = wid * (BPW * SEQ)
        pltpu.sync_copy(idx_hbm.at[pl.ds(ibase, BPW * SEQ)], idx_v)

        def fire(ci, rows, sem):
            for j in range(NGATHER):
                pltpu.async_copy(
                    table_hbm.at[idx_v.at[pl.ds(ci * IDX_PER_CHUNK + j * GW, GW)]],
                    rows.at[pl.ds(j * GW, GW)],
                    sem,
                )

        def drain(rows, sem):
            pltpu.make_async_copy(
                table_hbm.at[pl.ds(0, IDX_PER_CHUNK)], rows, sem).wait()

        def compute(ci, rows):
            @pl.loop(0, G)
            def _group(g):
                base = g * SEQ
                for c in range(D // LANES):
                    sl = pl.ds(c * LANES, LANES)
                    acc = rows[base, sl]
                    for j in range(1, SEQ):
                        acc = acc + rows[base + j, sl]
                    pool_v[g, sl] = acc

            pltpu.sync_copy(
                pool_v, out_hbm.at[pl.ds(wid * BPW + ci * G, G)])

        fire(0, rows_a, sem_a)

        @pl.loop(0, NCHUNK // 2)
        def _pair(cp):
            ci0 = 2 * cp
            fire(ci0 + 1, rows_b, sem_b)
            drain(rows_a, sem_a)
            compute(ci0, rows_a)

            @pl.when(cp < NCHUNK // 2 - 1)
            def _():
                fire(ci0 + 2, rows_a, sem_a)

            drain(rows_b, sem_b)
            compute(ci0 + 1, rows_b)

    return sc_kernel(idx_flat, table_wide)


def _tc_head(pooled, W, b_scaled):
    BB = 2048

    def body(x_ref, w_ref, b_ref, o_ref):
        z = lax.dot_general(
            x_ref[...], w_ref[...], (((1,), (0,)), ((), ())),
            preferred_element_type=jnp.float32,
            precision=lax.Precision.HIGHEST,
        )
        z = z + b_ref[0:1, :]
        s = jnp.sum(z * z, axis=1, keepdims=True)
        o_ref[...] = z * lax.rsqrt(s)

    return pl.pallas_call(
        body,
        grid=(B // BB,),
        in_specs=[
            pl.BlockSpec((BB, D), lambda i: (i, 0)),
            pl.BlockSpec((D, D), lambda i: (0, 0)),
            pl.BlockSpec((8, D), lambda i: (0, 0)),
        ],
        out_specs=pl.BlockSpec((BB, D), lambda i: (i, 0)),
        out_shape=jax.ShapeDtypeStruct((B, D), jnp.float32),
        compiler_params=pltpu.CompilerParams(
            dimension_semantics=("parallel",)),
    )(pooled, W, b_scaled)


def kernel(inputs, table, W, b):
    idx_flat = inputs.reshape(-1).astype(jnp.int32)
    table_wide = _tc_widen(table.T)
    pooled = _sc_pool(idx_flat, table_wide)
    b_scaled = jnp.broadcast_to((SEQ * b).reshape(1, D), (8, D))
    return _tc_head(pooled, W, b_scaled)


# speedup vs baseline: 1.1211x; 1.0018x over previous
import functools

import jax
import jax.numpy as jnp
from jax import lax
from jax.experimental import pallas as pl
from jax.experimental.pallas import tpu as pltpu
from jax.experimental.pallas import tpu_sc as plsc

NUM_EMB = 1000000
B = 16384
SEQ = 20
D = 64
LANES = 16
NC, NS = 2, 16
NW = NC * NS
BPW = B // NW
G = 16
NCHUNK = BPW // G
IDX_PER_CHUNK = G * SEQ
GW = 64
NGATHER = IDX_PER_CHUNK // GW


def _tc_widen(tableT):
    BLK = 28672

    def body(t_ref, o_ref):
        tt = jnp.transpose(t_ref[...], (1, 0))
        o_ref[...] = jnp.concatenate([tt, tt], axis=1)

    return pl.pallas_call(
        body,
        grid=(pl.cdiv(NUM_EMB, BLK),),
        in_specs=[pl.BlockSpec((D, BLK), lambda i: (0, i))],
        out_specs=pl.BlockSpec((BLK, 2 * D), lambda i: (i, 0)),
        out_shape=jax.ShapeDtypeStruct((NUM_EMB, 2 * D), jnp.float32),
        compiler_params=pltpu.CompilerParams(
            dimension_semantics=("parallel",)),
    )(tableT)


def _sc_pool(idx_flat, table_wide):
    mesh = plsc.VectorSubcoreMesh(core_axis_name="c", subcore_axis_name="s")

    @functools.partial(
        pl.kernel,
        mesh=mesh,
        out_type=jax.ShapeDtypeStruct((B, D), jnp.float32),
        scratch_types=[
            pltpu.VMEM((BPW * SEQ,), jnp.int32),
            pltpu.VMEM((IDX_PER_CHUNK, 2 * D), jnp.float32),
            pltpu.VMEM((IDX_PER_CHUNK, 2 * D), jnp.float32),
            pltpu.VMEM((G, D), jnp.float32),
            pltpu.SemaphoreType.DMA,
            pltpu.SemaphoreType.DMA,
        ],
    )
    def sc_kernel(idx_hbm, table_hbm, out_hbm, idx_v, rows_a, rows_b,
                  pool_v, sem_a, sem_b):
        wid = lax.axis_index("s") * NC + lax.axis_index("c")
        ibase = wid * (BPW * SEQ)
        pltpu.sync_copy(idx_hbm.at[pl.ds(ibase, BPW * SEQ)], idx_v)

        def fire(ci, rows, sem):
            for j in range(NGATHER):
                pltpu.async_copy(
                    table_hbm.at[idx_v.at[pl.ds(ci * IDX_PER_CHUNK + j * GW, GW)]],
                    rows.at[pl.ds(j * GW, GW)],
                    sem,
                )

        def drain(rows, sem):
            pltpu.make_async_copy(
                table_hbm.at[pl.ds(0, IDX_PER_CHUNK)], rows, sem).wait()

        def compute(ci, rows):
            @pl.loop(0, G)
            def _group(g):
                base = g * SEQ
                for c in range(D // LANES):
                    sl = pl.ds(c * LANES, LANES)
                    acc = rows[base, sl]
                    for j in range(1, SEQ):
                        acc = acc + rows[base + j, sl]
                    pool_v[g, sl] = acc

            pltpu.sync_copy(
                pool_v, out_hbm.at[pl.ds(wid * BPW + ci * G, G)])

        fire(0, rows_a, sem_a)

        @pl.loop(0, NCHUNK // 2)
        def _pair(cp):
            ci0 = 2 * cp
            fire(ci0 + 1, rows_b, sem_b)
            drain(rows_a, sem_a)
            compute(ci0, rows_a)

            @pl.when(cp < NCHUNK // 2 - 1)
            def _():
                fire(ci0 + 2, rows_a, sem_a)

            drain(rows_b, sem_b)
            compute(ci0 + 1, rows_b)

    return sc_kernel(idx_flat, table_wide)


def _tc_head(pooled, W, b_scaled):
    BB = 2048

    def body(x_ref, w_ref, b_ref, o_ref):
        z = lax.dot_general(
            x_ref[...], w_ref[...], (((1,), (0,)), ((), ())),
            preferred_element_type=jnp.float32,
            precision=lax.Precision.HIGHEST,
        )
        z = z + b_ref[0:1, :]
        s = jnp.sum(z * z, axis=1, keepdims=True)
        o_ref[...] = z * lax.rsqrt(s)

    return pl.pallas_call(
        body,
        grid=(B // BB,),
        in_specs=[
            pl.BlockSpec((BB, D), lambda i: (i, 0)),
            pl.BlockSpec((D, D), lambda i: (0, 0)),
            pl.BlockSpec((8, D), lambda i: (0, 0)),
        ],
        out_specs=pl.BlockSpec((BB, D), lambda i: (i, 0)),
        out_shape=jax.ShapeDtypeStruct((B, D), jnp.float32),
        compiler_params=pltpu.CompilerParams(
            dimension_semantics=("parallel",)),
    )(pooled, W, b_scaled)


def kernel(inputs, table, W, b):
    idx_flat = inputs.reshape(-1).astype(jnp.int32)
    table_wide = _tc_widen(table.T)
    pooled = _sc_pool(idx_flat, table_wide)
    b_scaled = jnp.broadcast_to((SEQ * b).reshape(1, D), (8, D))
    return _tc_head(pooled, W, b_scaled)
